# baseline (speedup 1.0000x reference)
import jax
import jax.numpy as jnp
from jax import lax
from jax.experimental import pallas as pl
from jax.experimental.pallas import tpu as pltpu
from jax.experimental.pallas import tpu_sc as plsc

BATCH = 16384
D = 128
NC = 2
NS = 16
NW = NC * NS
LANES = 16
COLS = D // LANES
ROWS_W = BATCH // NW
NCH = 2
ROWS_CH = ROWS_W // NCH


def _sc_body(label_hbm, x_hbm, scale_hbm, offset_hbm, out_hbm,
             idx_v, srow, orow, b0, b1,
             sem_g, sem_i0, sem_i1, sem_o0, sem_o1):
    wid = lax.axis_index("s") * NC + lax.axis_index("c")
    base = wid * ROWS_W
    bufs = (b0, b1)
    sem_i = (sem_i0, sem_i1)
    sem_o = (sem_o0, sem_o1)

    pltpu.sync_copy(label_hbm, idx_v)
    c_s = pltpu.async_copy(scale_hbm.at[idx_v], srow, sem_g)
    c_o = pltpu.async_copy(offset_hbm.at[idx_v], orow, sem_g)

    in_cp = [pltpu.async_copy(
        x_hbm.at[pl.ds(base + g * ROWS_CH, ROWS_CH)], bufs[g], sem_i[g])
        for g in range(NCH)]

    c_s.wait()
    c_o.wait()
    svals = [srow[0, pl.ds(j * LANES, LANES)] for j in range(COLS)]
    ovals = [orow[0, pl.ds(j * LANES, LANES)] for j in range(COLS)]

    out_cp = [None] * NCH
    for g in range(NCH):
        in_cp[g].wait()
        buf = bufs[g]

        def row(r, carry):
            for j in range(COLS):
                sl = pl.ds(j * LANES, LANES)
                buf[r, sl] = svals[j] * buf[r, sl] + ovals[j]
            return carry

        lax.fori_loop(0, ROWS_CH, row, 0)

        out_cp[g] = pltpu.async_copy(
            buf, out_hbm.at[pl.ds(base + g * ROWS_CH, ROWS_CH)], sem_o[g])

    for g in range(NCH):
        out_cp[g].wait()


def kernel(inputs, scale_table, offset_table, label):
    label_arr = jnp.asarray(label, dtype=jnp.int32).reshape((1,))
    mesh = plsc.VectorSubcoreMesh(core_axis_name="c", subcore_axis_name="s")
    return pl.kernel(
        _sc_body,
        out_type=jax.ShapeDtypeStruct((BATCH, D), jnp.float32),
        mesh=mesh,
        scratch_types=[
            pltpu.VMEM((1,), jnp.int32),
            pltpu.VMEM((1, D), jnp.float32),
            pltpu.VMEM((1, D), jnp.float32),
            pltpu.VMEM((ROWS_CH, D), jnp.float32),
            pltpu.VMEM((ROWS_CH, D), jnp.float32),
            pltpu.SemaphoreType.DMA,
            pltpu.SemaphoreType.DMA,
            pltpu.SemaphoreType.DMA,
            pltpu.SemaphoreType.DMA,
            pltpu.SemaphoreType.DMA,
        ],
    )(label_arr, inputs, scale_table, offset_table)


# speedup vs baseline: 1.0329x; 1.0329x over previous
import jax
import jax.numpy as jnp
from jax import lax
from jax.experimental import pallas as pl
from jax.experimental.pallas import tpu as pltpu
from jax.experimental.pallas import tpu_sc as plsc

BATCH = 16384
D = 128
NC = 2
NS = 16
NW = NC * NS
LANES = 16
COLS = D // LANES
ROWS_W = BATCH // NW
NCH = 8
ROWS_CH = ROWS_W // NCH


def _sc_body(label_hbm, x_hbm, scale_hbm, offset_hbm, out_hbm,
             idx_v, srow, orow, b0, b1, b2, b3, b4, b5, b6, b7,
             sem_g, si0, si1, si2, si3, si4, si5, si6, si7,
             so0, so1, so2, so3, so4, so5, so6, so7):
    wid = lax.axis_index("s") * NC + lax.axis_index("c")
    base = wid * ROWS_W
    bufs = (b0, b1, b2, b3, b4, b5, b6, b7)
    sem_i = (si0, si1, si2, si3, si4, si5, si6, si7)
    sem_o = (so0, so1, so2, so3, so4, so5, so6, so7)

    pltpu.sync_copy(label_hbm, idx_v)
    c_s = pltpu.async_copy(scale_hbm.at[idx_v], srow, sem_g)
    c_o = pltpu.async_copy(offset_hbm.at[idx_v], orow, sem_g)

    in_cp = [pltpu.async_copy(
        x_hbm.at[pl.ds(base + g * ROWS_CH, ROWS_CH)], bufs[g], sem_i[g])
        for g in range(NCH)]

    c_s.wait()
    c_o.wait()
    svals = [srow[0, pl.ds(j * LANES, LANES)] for j in range(COLS)]
    ovals = [orow[0, pl.ds(j * LANES, LANES)] for j in range(COLS)]

    out_cp = [None] * NCH
    for g in range(NCH):
        in_cp[g].wait()
        buf = bufs[g]

        def row(r, carry):
            for j in range(COLS):
                sl = pl.ds(j * LANES, LANES)
                buf[r, sl] = svals[j] * buf[r, sl] + ovals[j]
            return carry

        lax.fori_loop(0, ROWS_CH, row, 0)

        out_cp[g] = pltpu.async_copy(
            buf, out_hbm.at[pl.ds(base + g * ROWS_CH, ROWS_CH)], sem_o[g])

    for g in range(NCH):
        out_cp[g].wait()


def kernel(inputs, scale_table, offset_table, label):
    label_arr = jnp.asarray(label, dtype=jnp.int32).reshape((1,))
    mesh = plsc.VectorSubcoreMesh(core_axis_name="c", subcore_axis_name="s")
    return pl.kernel(
        _sc_body,
        out_type=jax.ShapeDtypeStruct((BATCH, D), jnp.float32),
        mesh=mesh,
        scratch_types=[
            pltpu.VMEM((1,), jnp.int32),
            pltpu.VMEM((1, D), jnp.float32),
            pltpu.VMEM((1, D), jnp.float32),
            pltpu.VMEM((ROWS_CH, D), jnp.float32),
            pltpu.VMEM((ROWS_CH, D), jnp.float32),
            pltpu.VMEM((ROWS_CH, D), jnp.float32),
            pltpu.VMEM((ROWS_CH, D), jnp.float32),
            pltpu.VMEM((ROWS_CH, D), jnp.float32),
            pltpu.VMEM((ROWS_CH, D), jnp.float32),
            pltpu.VMEM((ROWS_CH, D), jnp.float32),
            pltpu.VMEM((ROWS_CH, D), jnp.float32),
        ] + [pltpu.SemaphoreType.DMA] * 17,
    )(label_arr, inputs, scale_table, offset_table)
